# baseline (speedup 1.0000x reference)
import functools

import jax
import jax.numpy as jnp
from jax.experimental import pallas as pl
from jax.experimental.pallas import tpu as pltpu
from jax.experimental.pallas import tpu_sc as plsc

T = 2048
D = 1024
F = 1024
E = 8
K = 2
TM = 256
TR = 256
G = (T * K) // TR + E
P = G * TR

NC = 2
NS = 16
NW = NC * NS
TPW = T // NW


def _router_kernel(x_ref, Wr_ref, rb_ref,
                   i1_ref, i2_ref, w1_ref, w2_ref, cnt_ref):
    t = pl.program_id(0)
    xt = x_ref[...]

    logits = jnp.dot(xt, Wr_ref[...], preferred_element_type=jnp.float32)
    logits = logits - jnp.max(logits, axis=-1, keepdims=True)
    ex = jnp.exp(logits)
    scores = ex / jnp.sum(ex, axis=-1, keepdims=True)
    sel = scores + rb_ref[...]
    iota_e = jax.lax.broadcasted_iota(jnp.int32, (TM, E), 1)

    m1 = jnp.max(sel, axis=-1, keepdims=True)
    i1 = jnp.min(jnp.where(sel == m1, iota_e, E), axis=-1, keepdims=True)
    sel2 = jnp.where(iota_e == i1, -jnp.inf, sel)
    m2 = jnp.max(sel2, axis=-1, keepdims=True)
    i2 = jnp.min(jnp.where(sel2 == m2, iota_e, E), axis=-1, keepdims=True)

    g1 = jnp.sum(jnp.where(iota_e == i1, scores, 0.0), axis=-1, keepdims=True)
    g2 = jnp.sum(jnp.where(iota_e == i2, scores, 0.0), axis=-1, keepdims=True)
    denom = g1 + g2 + 1e-9
    i1_ref[...] = i1
    i2_ref[...] = i2
    w1_ref[...] = g1 / denom
    w2_ref[...] = g2 / denom

    tile_cnt = (jnp.sum((iota_e == i1).astype(jnp.int32), axis=0, keepdims=True)
                + jnp.sum((iota_e == i2).astype(jnp.int32), axis=0, keepdims=True))

    @pl.when(t == 0)
    def _():
        cnt_ref[...] = tile_cnt

    @pl.when(t != 0)
    def _():
        cnt_ref[...] += tile_cnt


def _router(xf, Wr, rb2):
    return pl.pallas_call(
        _router_kernel,
        grid=(T // TM,),
        in_specs=[
            pl.BlockSpec((TM, D), lambda t: (t, 0)),
            pl.BlockSpec((D, E), lambda t: (0, 0)),
            pl.BlockSpec((1, E), lambda t: (0, 0)),
        ],
        out_specs=[
            pl.BlockSpec((TM, 1), lambda t: (t, 0)),
            pl.BlockSpec((TM, 1), lambda t: (t, 0)),
            pl.BlockSpec((TM, 1), lambda t: (t, 0)),
            pl.BlockSpec((TM, 1), lambda t: (t, 0)),
            pl.BlockSpec((1, E), lambda t: (0, 0)),
        ],
        out_shape=[
            jax.ShapeDtypeStruct((T, 1), jnp.int32),
            jax.ShapeDtypeStruct((T, 1), jnp.int32),
            jax.ShapeDtypeStruct((T, 1), jnp.float32),
            jax.ShapeDtypeStruct((T, 1), jnp.float32),
            jax.ShapeDtypeStruct((1, E), jnp.int32),
        ],
        compiler_params=pltpu.CompilerParams(
            dimension_semantics=("arbitrary",),
        ),
    )(xf, Wr, rb2)


def _shared_kernel(x_ref, Ws1_ref, Ws2_ref, sh_ref, w1b_ref, w2b_ref):
    t = pl.program_id(0)

    @pl.when(t == 0)
    def _():
        w1b_ref[...] = Ws1_ref[...].astype(jnp.bfloat16)
        w2b_ref[...] = Ws2_ref[...].astype(jnp.bfloat16)

    xt = x_ref[...].astype(jnp.bfloat16)
    hs = jnp.dot(xt, w1b_ref[...], preferred_element_type=jnp.float32)
    hs = hs * jax.nn.sigmoid(hs)
    sh_ref[...] = jnp.dot(hs.astype(jnp.bfloat16), w2b_ref[...],
                          preferred_element_type=jnp.float32)


def _shared_ffn(xf, Ws1, Ws2):
    return pl.pallas_call(
        _shared_kernel,
        grid=(T // TM,),
        in_specs=[
            pl.BlockSpec((TM, D), lambda t: (t, 0)),
            pl.BlockSpec((D, F), lambda t: (0, 0)),
            pl.BlockSpec((F, D), lambda t: (0, 0)),
        ],
        out_specs=pl.BlockSpec((TM, D), lambda t: (t, 0)),
        out_shape=jax.ShapeDtypeStruct((T, D), jnp.float32),
        scratch_shapes=[
            pltpu.VMEM((D, F), jnp.bfloat16),
            pltpu.VMEM((F, D), jnp.bfloat16),
        ],
        compiler_params=pltpu.CompilerParams(
            dimension_semantics=("arbitrary",),
        ),
    )(xf, Ws1, Ws2)


def _sc_dispatch(xf, p1m, p2m):
    mesh = plsc.VectorSubcoreMesh(core_axis_name="c", subcore_axis_name="s")

    @functools.partial(
        pl.kernel,
        out_type=jax.ShapeDtypeStruct((P, D), jnp.float32),
        mesh=mesh,
        scratch_types=[
            pltpu.VMEM((TPW,), jnp.int32),
            pltpu.VMEM((TPW,), jnp.int32),
            pltpu.VMEM((TPW, D), jnp.float32),
            pltpu.SemaphoreType.DMA,
            pltpu.SemaphoreType.DMA,
            pltpu.SemaphoreType.DMA,
        ],
    )
    def k(x_hbm, p1_hbm, p2_hbm, o_hbm, i1_v, i2_v, rows_v, s0, s1, s2):
        wid = jax.lax.axis_index("s") * NC + jax.lax.axis_index("c")
        c1 = pltpu.async_copy(p1_hbm.at[wid], i1_v, s0)
        c2 = pltpu.async_copy(p2_hbm.at[wid], i2_v, s1)
        cr = pltpu.async_copy(x_hbm.at[pl.ds(wid * TPW, TPW)], rows_v, s2)
        c1.wait()
        c2.wait()
        cr.wait()
        w1 = pltpu.async_copy(rows_v, o_hbm.at[i1_v], s0)
        w2 = pltpu.async_copy(rows_v, o_hbm.at[i2_v], s1)
        w1.wait()
        w2.wait()

    return k(xf, p1m, p2m)


def _sc_gather(table, idx, n_rows, gw):
    d = table.shape[1]
    b_per_w = n_rows // NW
    chunks = b_per_w // gw
    assert chunks * gw == b_per_w
    mesh = plsc.VectorSubcoreMesh(core_axis_name="c", subcore_axis_name="s")

    @functools.partial(
        pl.kernel,
        out_type=jax.ShapeDtypeStruct((n_rows, d), table.dtype),
        mesh=mesh,
        scratch_types=[
            pltpu.VMEM((b_per_w,), jnp.int32),
            pltpu.VMEM((gw, d), table.dtype),
            pltpu.VMEM((gw, d), table.dtype),
            pltpu.SemaphoreType.DMA,
            pltpu.SemaphoreType.DMA,
            pltpu.SemaphoreType.DMA,
            pltpu.SemaphoreType.DMA,
        ],
    )
    def k(x_hbm, i_hbm, o_hbm, idx_v, r0, r1, gs0, gs1, ss0, ss1):
        wid = jax.lax.axis_index("s") * NC + jax.lax.axis_index("c")
        base = wid * b_per_w
        pltpu.sync_copy(i_hbm.at[pl.ds(base, b_per_w)], idx_v)
        bufs, gsem, ssem = [r0, r1], [gs0, gs1], [ss0, ss1]
        gh = [None] * chunks
        sh = [None] * chunks
        for j in range(chunks):
            b = j % 2
            if j >= 2:
                sh[j - 2].wait()
            gh[j] = pltpu.async_copy(
                x_hbm.at[idx_v.at[pl.ds(j * gw, gw)]], bufs[b], gsem[b])
            if j >= 1:
                gh[j - 1].wait()
                sh[j - 1] = pltpu.async_copy(
                    bufs[(j - 1) % 2],
                    o_hbm.at[pl.ds(base + (j - 1) * gw, gw)],
                    ssem[(j - 1) % 2])
        gh[chunks - 1].wait()
        sh[chunks - 1] = pltpu.async_copy(
            bufs[(chunks - 1) % 2],
            o_hbm.at[pl.ds(base + (chunks - 1) * gw, gw)],
            ssem[(chunks - 1) % 2])
        for j in range(max(0, chunks - 2), chunks):
            sh[j].wait()

    return k(table, idx)


def _grouped_ffn_kernel(s_ref, x_ref, W1_ref, W2_ref, y_ref, w1b_ref, w2b_ref):
    g = pl.program_id(0)

    @pl.when(s_ref[2, g] == 1)
    def _():
        @pl.when(s_ref[3, g] == 1)
        def _():
            w1b_ref[...] = W1_ref[0].astype(jnp.bfloat16)
            w2b_ref[...] = W2_ref[0].astype(jnp.bfloat16)

        xt = x_ref[...].astype(jnp.bfloat16)
        h = jnp.dot(xt, w1b_ref[...], preferred_element_type=jnp.float32)
        h = h * jax.nn.sigmoid(h)
        y_ref[...] = jnp.dot(h.astype(jnp.bfloat16), w2b_ref[...],
                             preferred_element_type=jnp.float32)


def _grouped_ffn(x_sorted, W1, W2, meta):
    grid_spec = pltpu.PrefetchScalarGridSpec(
        num_scalar_prefetch=1,
        grid=(G,),
        in_specs=[
            pl.BlockSpec((TR, D), lambda g, s: (s[0, g], 0)),
            pl.BlockSpec((1, D, F), lambda g, s: (s[1, g], 0, 0)),
            pl.BlockSpec((1, F, D), lambda g, s: (s[1, g], 0, 0)),
        ],
        out_specs=pl.BlockSpec((TR, D), lambda g, s: (s[0, g], 0)),
        scratch_shapes=[
            pltpu.VMEM((D, F), jnp.bfloat16),
            pltpu.VMEM((F, D), jnp.bfloat16),
        ],
    )
    return pl.pallas_call(
        _grouped_ffn_kernel,
        grid_spec=grid_spec,
        out_shape=jax.ShapeDtypeStruct((P, D), jnp.float32),
        compiler_params=pltpu.CompilerParams(
            dimension_semantics=("arbitrary",),
        ),
    )(meta, x_sorted, W1, W2)


def _combine_kernel(sh_ref, w1_ref, w2_ref, ya_ref, yb_ref, o_ref):
    o_ref[...] = (sh_ref[...] + w1_ref[...] * ya_ref[...]
                  + w2_ref[...] * yb_ref[...])


def _combine(shared, w1, w2, y12):
    return pl.pallas_call(
        _combine_kernel,
        grid=(T // TM,),
        in_specs=[
            pl.BlockSpec((TM, D), lambda t: (t, 0)),
            pl.BlockSpec((TM, 1), lambda t: (t, 0)),
            pl.BlockSpec((TM, 1), lambda t: (t, 0)),
            pl.BlockSpec((TM, D), lambda t: (t, 0)),
            pl.BlockSpec((TM, D), lambda t: (t + T // TM, 0)),
        ],
        out_specs=pl.BlockSpec((TM, D), lambda t: (t, 0)),
        out_shape=jax.ShapeDtypeStruct((T, D), jnp.float32),
        compiler_params=pltpu.CompilerParams(
            dimension_semantics=("parallel",),
        ),
    )(shared, w1, w2, y12, y12)


def kernel(x, Ws1, Ws2, W1, W2, Wr, rb):
    b, s, d = x.shape
    xf = x.reshape(s, d)
    rb2 = rb.reshape(1, E)

    i1, i2, w1, w2, cnt = _router(xf, Wr, rb2)
    shared = _shared_ffn(xf, Ws1, Ws2)

    ii = jnp.concatenate([i1, i2], axis=1)
    ef = ii.reshape(-1)
    counts = cnt.reshape(E)
    tiles_e = (counts + TR - 1) // TR
    cum_tiles = jnp.cumsum(tiles_e)
    total_tiles = cum_tiles[-1]
    seg_start = (cum_tiles - tiles_e) * TR

    oh = (ef[:, None] == jnp.arange(E, dtype=jnp.int32)[None, :]).astype(jnp.int32)
    rank = jnp.sum((jnp.cumsum(oh, axis=0) - oh) * oh, axis=1)
    pos = jnp.sum(seg_start[None, :] * oh, axis=1) + rank

    ga = jnp.arange(G, dtype=jnp.int32)
    gg = jnp.minimum(ga, total_tiles - 1)
    eidp = jnp.sum((gg[:, None] >= cum_tiles[None, :]).astype(jnp.int32), axis=1)
    vld = (ga < total_tiles).astype(jnp.int32)
    newe = jnp.concatenate([jnp.ones((1,), jnp.int32),
                            (eidp[1:] != eidp[:-1]).astype(jnp.int32)]) * vld
    meta = jnp.stack([gg, eidp, vld, newe], axis=0)

    posm = pos.reshape(T, K)
    p1m = posm[:, 0].reshape(NW, TPW)
    p2m = posm[:, 1].reshape(NW, TPW)
    p12 = jnp.concatenate([posm[:, 0], posm[:, 1]], axis=0)

    x_sorted = _sc_dispatch(xf, p1m, p2m)
    y = _grouped_ffn(x_sorted, W1, W2, meta)
    y12 = _sc_gather(y, p12, T * K, 32)
    out = _combine(shared, w1, w2, y12)

    return out.reshape(b, s, d), counts


# speedup vs baseline: 1.0382x; 1.0382x over previous
import functools

import jax
import jax.numpy as jnp
from jax.experimental import pallas as pl
from jax.experimental.pallas import tpu as pltpu
from jax.experimental.pallas import tpu_sc as plsc

T = 2048
D = 1024
F = 1024
E = 8
K = 2
TM = 256
TR = 256
G = (T * K) // TR + E
P = G * TR

NC = 2
NS = 16
NW = NC * NS
TPW = T // NW


def _router_kernel(x_ref, Wr_ref, rb_ref,
                   i1_ref, i2_ref, w1_ref, w2_ref, r1_ref, r2_ref, cnt_ref):
    t = pl.program_id(0)
    xt = x_ref[...]

    logits = jnp.dot(xt, Wr_ref[...], preferred_element_type=jnp.float32)
    logits = logits - jnp.max(logits, axis=-1, keepdims=True)
    ex = jnp.exp(logits)
    scores = ex / jnp.sum(ex, axis=-1, keepdims=True)
    sel = scores + rb_ref[...]
    iota_e = jax.lax.broadcasted_iota(jnp.int32, (TM, E), 1)

    m1 = jnp.max(sel, axis=-1, keepdims=True)
    i1 = jnp.min(jnp.where(sel == m1, iota_e, E), axis=-1, keepdims=True)
    sel2 = jnp.where(iota_e == i1, -jnp.inf, sel)
    m2 = jnp.max(sel2, axis=-1, keepdims=True)
    i2 = jnp.min(jnp.where(sel2 == m2, iota_e, E), axis=-1, keepdims=True)

    g1 = jnp.sum(jnp.where(iota_e == i1, scores, 0.0), axis=-1, keepdims=True)
    g2 = jnp.sum(jnp.where(iota_e == i2, scores, 0.0), axis=-1, keepdims=True)
    denom = g1 + g2 + 1e-9
    i1_ref[...] = i1
    i2_ref[...] = i2
    w1_ref[...] = g1 / denom
    w2_ref[...] = g2 / denom

    oh1 = (iota_e == i1).astype(jnp.float32)
    oh2 = (iota_e == i2).astype(jnp.float32)
    oh12 = oh1 + oh2
    ir = jax.lax.broadcasted_iota(jnp.int32, (TM, TM), 0)
    ic = jax.lax.broadcasted_iota(jnp.int32, (TM, TM), 1)
    tri = (ir > ic).astype(jnp.float32)
    pre = jnp.dot(tri, oh12, preferred_element_type=jnp.float32)
    carry = jnp.where(t == 0, 0, cnt_ref[...]).astype(jnp.float32)
    prefix = pre + carry
    r1_ref[...] = jnp.sum(prefix * oh1, axis=-1, keepdims=True).astype(jnp.int32)
    r2_ref[...] = jnp.sum(prefix * oh2, axis=-1, keepdims=True).astype(jnp.int32)

    tile_cnt = jnp.sum(oh12, axis=0, keepdims=True).astype(jnp.int32)

    @pl.when(t == 0)
    def _():
        cnt_ref[...] = tile_cnt

    @pl.when(t != 0)
    def _():
        cnt_ref[...] += tile_cnt


def _router(xf, Wr, rb2):
    return pl.pallas_call(
        _router_kernel,
        grid=(T // TM,),
        in_specs=[
            pl.BlockSpec((TM, D), lambda t: (t, 0)),
            pl.BlockSpec((D, E), lambda t: (0, 0)),
            pl.BlockSpec((1, E), lambda t: (0, 0)),
        ],
        out_specs=[
            pl.BlockSpec((TM, 1), lambda t: (t, 0)),
            pl.BlockSpec((TM, 1), lambda t: (t, 0)),
            pl.BlockSpec((TM, 1), lambda t: (t, 0)),
            pl.BlockSpec((TM, 1), lambda t: (t, 0)),
            pl.BlockSpec((TM, 1), lambda t: (t, 0)),
            pl.BlockSpec((TM, 1), lambda t: (t, 0)),
            pl.BlockSpec((1, E), lambda t: (0, 0)),
        ],
        out_shape=[
            jax.ShapeDtypeStruct((T, 1), jnp.int32),
            jax.ShapeDtypeStruct((T, 1), jnp.int32),
            jax.ShapeDtypeStruct((T, 1), jnp.float32),
            jax.ShapeDtypeStruct((T, 1), jnp.float32),
            jax.ShapeDtypeStruct((T, 1), jnp.int32),
            jax.ShapeDtypeStruct((T, 1), jnp.int32),
            jax.ShapeDtypeStruct((1, E), jnp.int32),
        ],
        compiler_params=pltpu.CompilerParams(
            dimension_semantics=("arbitrary",),
        ),
    )(xf, Wr, rb2)


def _shared_kernel(x_ref, Ws1_ref, Ws2_ref, sh_ref):
    xt = x_ref[...]
    hs = jnp.dot(xt, Ws1_ref[...], preferred_element_type=jnp.float32)
    hs = hs * jax.nn.sigmoid(hs)
    sh_ref[...] = jnp.dot(hs, Ws2_ref[...], preferred_element_type=jnp.float32)


def _shared_ffn(xf, Ws1, Ws2):
    return pl.pallas_call(
        _shared_kernel,
        grid=(T // TM,),
        in_specs=[
            pl.BlockSpec((TM, D), lambda t: (t, 0)),
            pl.BlockSpec((D, F), lambda t: (0, 0)),
            pl.BlockSpec((F, D), lambda t: (0, 0)),
        ],
        out_specs=pl.BlockSpec((TM, D), lambda t: (t, 0)),
        out_shape=jax.ShapeDtypeStruct((T, D), jnp.float32),
        compiler_params=pltpu.CompilerParams(
            dimension_semantics=("arbitrary",),
        ),
    )(xf, Ws1, Ws2)


def _sc_dispatch(xf, p1m, p2m):
    mesh = plsc.VectorSubcoreMesh(core_axis_name="c", subcore_axis_name="s")

    @functools.partial(
        pl.kernel,
        out_type=jax.ShapeDtypeStruct((P, D), jnp.float32),
        mesh=mesh,
        scratch_types=[
            pltpu.VMEM((TPW,), jnp.int32),
            pltpu.VMEM((TPW,), jnp.int32),
            pltpu.VMEM((TPW, D), jnp.float32),
            pltpu.SemaphoreType.DMA,
            pltpu.SemaphoreType.DMA,
            pltpu.SemaphoreType.DMA,
        ],
    )
    def k(x_hbm, p1_hbm, p2_hbm, o_hbm, i1_v, i2_v, rows_v, s0, s1, s2):
        wid = jax.lax.axis_index("s") * NC + jax.lax.axis_index("c")
        c1 = pltpu.async_copy(p1_hbm.at[wid], i1_v, s0)
        c2 = pltpu.async_copy(p2_hbm.at[wid], i2_v, s1)
        cr = pltpu.async_copy(x_hbm.at[pl.ds(wid * TPW, TPW)], rows_v, s2)
        c1.wait()
        c2.wait()
        cr.wait()
        w1 = pltpu.async_copy(rows_v, o_hbm.at[i1_v], s0)
        w2 = pltpu.async_copy(rows_v, o_hbm.at[i2_v], s1)
        w1.wait()
        w2.wait()

    return k(xf, p1m, p2m)


def _sc_gather(table, idx, n_rows, gw):
    d = table.shape[1]
    b_per_w = n_rows // NW
    chunks = b_per_w // gw
    assert chunks * gw == b_per_w
    mesh = plsc.VectorSubcoreMesh(core_axis_name="c", subcore_axis_name="s")

    @functools.partial(
        pl.kernel,
        out_type=jax.ShapeDtypeStruct((n_rows, d), table.dtype),
        mesh=mesh,
        scratch_types=[
            pltpu.VMEM((b_per_w,), jnp.int32),
            pltpu.VMEM((gw, d), table.dtype),
            pltpu.VMEM((gw, d), table.dtype),
            pltpu.SemaphoreType.DMA,
            pltpu.SemaphoreType.DMA,
            pltpu.SemaphoreType.DMA,
            pltpu.SemaphoreType.DMA,
        ],
    )
    def k(x_hbm, i_hbm, o_hbm, idx_v, r0, r1, gs0, gs1, ss0, ss1):
        wid = jax.lax.axis_index("s") * NC + jax.lax.axis_index("c")
        base = wid * b_per_w
        pltpu.sync_copy(i_hbm.at[pl.ds(base, b_per_w)], idx_v)
        bufs, gsem, ssem = [r0, r1], [gs0, gs1], [ss0, ss1]
        gh = [None] * chunks
        sh = [None] * chunks
        for j in range(chunks):
            b = j % 2
            if j >= 2:
                sh[j - 2].wait()
            gh[j] = pltpu.async_copy(
                x_hbm.at[idx_v.at[pl.ds(j * gw, gw)]], bufs[b], gsem[b])
            if j >= 1:
                gh[j - 1].wait()
                sh[j - 1] = pltpu.async_copy(
                    bufs[(j - 1) % 2],
                    o_hbm.at[pl.ds(base + (j - 1) * gw, gw)],
                    ssem[(j - 1) % 2])
        gh[chunks - 1].wait()
        sh[chunks - 1] = pltpu.async_copy(
            bufs[(chunks - 1) % 2],
            o_hbm.at[pl.ds(base + (chunks - 1) * gw, gw)],
            ssem[(chunks - 1) % 2])
        for j in range(max(0, chunks - 2), chunks):
            sh[j].wait()

    return k(table, idx)


def _grouped_ffn_kernel(s_ref, x_ref, W1_ref, W2_ref, y_ref):
    g = pl.program_id(0)

    @pl.when(s_ref[2, g] == 1)
    def _():
        xt = x_ref[...]
        h = jnp.dot(xt, W1_ref[0], preferred_element_type=jnp.float32)
        h = h * jax.nn.sigmoid(h)
        y_ref[...] = jnp.dot(h, W2_ref[0], preferred_element_type=jnp.float32)


def _grouped_ffn(x_sorted, W1, W2, meta):
    grid_spec = pltpu.PrefetchScalarGridSpec(
        num_scalar_prefetch=1,
        grid=(G,),
        in_specs=[
            pl.BlockSpec((TR, D), lambda g, s: (s[0, g], 0)),
            pl.BlockSpec((1, D, F), lambda g, s: (s[1, g], 0, 0)),
            pl.BlockSpec((1, F, D), lambda g, s: (s[1, g], 0, 0)),
        ],
        out_specs=pl.BlockSpec((TR, D), lambda g, s: (s[0, g], 0)),
    )
    return pl.pallas_call(
        _grouped_ffn_kernel,
        grid_spec=grid_spec,
        out_shape=jax.ShapeDtypeStruct((P, D), jnp.float32),
        compiler_params=pltpu.CompilerParams(
            dimension_semantics=("arbitrary",),
        ),
    )(meta, x_sorted, W1, W2)


def _combine_kernel(sh_ref, w1_ref, w2_ref, ya_ref, yb_ref, o_ref):
    o_ref[...] = (sh_ref[...] + w1_ref[...] * ya_ref[...]
                  + w2_ref[...] * yb_ref[...])


def _combine(shared, w1, w2, y12):
    return pl.pallas_call(
        _combine_kernel,
        grid=(T // TM,),
        in_specs=[
            pl.BlockSpec((TM, D), lambda t: (t, 0)),
            pl.BlockSpec((TM, 1), lambda t: (t, 0)),
            pl.BlockSpec((TM, 1), lambda t: (t, 0)),
            pl.BlockSpec((TM, D), lambda t: (t, 0)),
            pl.BlockSpec((TM, D), lambda t: (t + T // TM, 0)),
        ],
        out_specs=pl.BlockSpec((TM, D), lambda t: (t, 0)),
        out_shape=jax.ShapeDtypeStruct((T, D), jnp.float32),
        compiler_params=pltpu.CompilerParams(
            dimension_semantics=("parallel",),
        ),
    )(shared, w1, w2, y12, y12)


def kernel(x, Ws1, Ws2, W1, W2, Wr, rb):
    b, s, d = x.shape
    xf = x.reshape(s, d)
    rb2 = rb.reshape(1, E)

    i1, i2, w1, w2, r1, r2, cnt = _router(xf, Wr, rb2)
    shared = _shared_ffn(xf, Ws1, Ws2)

    counts = cnt.reshape(E)
    tiles_e = (counts + TR - 1) // TR
    cum_tiles = jnp.cumsum(tiles_e)
    total_tiles = cum_tiles[-1]
    seg_start = (cum_tiles - tiles_e) * TR

    er = jnp.arange(E, dtype=jnp.int32)[None, :]
    pos1 = jnp.sum(jnp.where(i1 == er, seg_start[None, :], 0), axis=1) + r1[:, 0]
    pos2 = jnp.sum(jnp.where(i2 == er, seg_start[None, :], 0), axis=1) + r2[:, 0]

    ga = jnp.arange(G, dtype=jnp.int32)
    gg = jnp.minimum(ga, total_tiles - 1)
    eidp = jnp.sum((gg[:, None] >= cum_tiles[None, :]).astype(jnp.int32), axis=1)
    vld = (ga < total_tiles).astype(jnp.int32)
    meta = jnp.stack([gg, eidp, vld], axis=0)

    p1m = pos1.reshape(NW, TPW)
    p2m = pos2.reshape(NW, TPW)
    p12 = jnp.concatenate([pos1, pos2], axis=0)

    x_sorted = _sc_dispatch(xf, p1m, p2m)
    y = _grouped_ffn(x_sorted, W1, W2, meta)
    y12 = _sc_gather(y, p12, T * K, 32)
    out = _combine(shared, w1, w2, y12)

    return out.reshape(b, s, d), counts
